# baseline (speedup 1.0000x reference)
import jax
import jax.numpy as jnp
from jax import lax
from jax.experimental import pallas as pl
from jax.experimental.pallas import tpu as pltpu
from jax.experimental.pallas import tpu_sc as plsc

D = 64
S = 200
BT = 4096
SCALE = 8.0
LANES = 16

NW = 32
ROWS_W = BT // NW
CHB = 2
N_CHUNKS = ROWS_W // CHB
NBUF = 2
FULL_G = S // LANES


def _emb_kernel(tok_hbm, table_hbm, out_hbm, idx_v, rows_v,
                sem_g0, sem_g1, sem_o0, sem_o1):
    nc = 2
    wid = lax.axis_index("s") * nc + lax.axis_index("c")
    base = wid * ROWS_W
    sem_g = (sem_g0, sem_g1)
    sem_o = (sem_o0, sem_o1)

    def fire(g, b):
        rb0 = base + g * CHB
        pltpu.sync_copy(tok_hbm.at[pl.ds(rb0, CHB)], idx_v.at[b])
        for r in range(CHB):
            pltpu.async_copy(
                table_hbm.at[idx_v.at[b, r, pl.ds(0, 128)]],
                rows_v.at[b, r, pl.ds(0, 128)],
                sem_g[b])
            pltpu.async_copy(
                table_hbm.at[idx_v.at[b, r, pl.ds(128, S - 128)]],
                rows_v.at[b, r, pl.ds(128, S - 128)],
                sem_g[b])

    def wait_gathers(b):
        pltpu.make_async_copy(
            out_hbm.at[pl.ds(0, CHB)], rows_v.at[b], sem_g[b]).wait()

    def wait_out(b):
        pltpu.make_async_copy(
            rows_v.at[b], out_hbm.at[pl.ds(0, CHB)], sem_o[b]).wait()

    def scale_rows(b, r, t, jr0, lane0, nrows):
        for i in range(nrows):
            jr = jr0 + i
            s = jnp.where(t[lane0 + i] == 0,
                          jnp.float32(0.0), jnp.float32(SCALE))
            sp = jnp.full((LANES,), s, jnp.float32)
            for c in range(D // LANES):
                x = rows_v[b, r, jr, pl.ds(c * LANES, LANES)]
                rows_v[b, r, jr, pl.ds(c * LANES, LANES)] = x * sp

    def scale_chunk(b):
        for r in range(CHB):
            def grp_body(q, c2):
                t = idx_v[b, r, pl.ds(q * LANES, LANES)]
                scale_rows(b, r, t, q * LANES, 0, LANES)
                return c2

            lax.fori_loop(0, FULL_G, grp_body, 0)
            tail = S - FULL_G * LANES
            t = idx_v[b, r, pl.ds(S - LANES, LANES)]
            scale_rows(b, r, t, S - tail, LANES - tail, tail)

    def put(g, b):
        pltpu.async_copy(rows_v.at[b], out_hbm.at[pl.ds(base + g * CHB, CHB)],
                         sem_o[b])

    fire(0, 0)

    def chunk_body(g, carry):
        def stage(bb):
            @pl.when(g + 1 < N_CHUNKS)
            def _pref():
                @pl.when(g >= 1)
                def _w():
                    wait_out(1 - bb)
                fire(g + 1, 1 - bb)

            wait_gathers(bb)
            scale_chunk(bb)
            put(g, bb)

        @pl.when(lax.rem(g, 2) == 0)
        def _b0():
            stage(0)

        @pl.when(lax.rem(g, 2) == 1)
        def _b1():
            stage(1)

        return carry

    lax.fori_loop(0, N_CHUNKS, chunk_body, 0)
    wait_out((N_CHUNKS - 1) % 2)
    wait_out(N_CHUNKS % 2)


def kernel(inp_tokens, emb_table):
    mesh = plsc.VectorSubcoreMesh(core_axis_name="c", subcore_axis_name="s")
    run = pl.kernel(
        _emb_kernel,
        mesh=mesh,
        out_type=jax.ShapeDtypeStruct((BT, S, D), jnp.float32),
        scratch_types=[
            pltpu.VMEM((NBUF, CHB, S), jnp.int32),
            pltpu.VMEM((NBUF, CHB, S, D), jnp.float32),
            pltpu.SemaphoreType.DMA,
            pltpu.SemaphoreType.DMA,
            pltpu.SemaphoreType.DMA,
            pltpu.SemaphoreType.DMA,
        ],
        compiler_params=pltpu.CompilerParams(use_tc_tiling_on_sc=False),
    )
    return run(inp_tokens, emb_table)


# speedup vs baseline: 1.2262x; 1.2262x over previous
import jax
import jax.numpy as jnp
from jax import lax
from jax.experimental import pallas as pl
from jax.experimental.layout import Format, Layout, with_layout_constraint
from jax.experimental.pallas import tpu as pltpu
from jax.experimental.pallas import tpu_sc as plsc

D = 64
S = 200
BT = 4096
SCALE = 8.0
LANES = 16

NW = 32
ROWS_W = BT // NW
CHB = 2
N_CHUNKS = ROWS_W // CHB
NBUF = 2
FULL_G = S // LANES


def _emb_kernel(tok_hbm, table_hbm, out_hbm, idx_v, rows_v,
                sem_g0, sem_g1, sem_o0, sem_o1):
    nc = 2
    wid = lax.axis_index("s") * nc + lax.axis_index("c")
    base = wid * ROWS_W
    sem_g = (sem_g0, sem_g1)
    sem_o = (sem_o0, sem_o1)

    def fire(g, b):
        rb0 = base + g * CHB
        pltpu.sync_copy(tok_hbm.at[pl.ds(rb0, CHB)], idx_v.at[b])
        for r in range(CHB):
            pltpu.async_copy(
                table_hbm.at[idx_v.at[b, r, pl.ds(0, 128)]],
                rows_v.at[b, r, pl.ds(0, 128)],
                sem_g[b])
            pltpu.async_copy(
                table_hbm.at[idx_v.at[b, r, pl.ds(128, S - 128)]],
                rows_v.at[b, r, pl.ds(128, S - 128)],
                sem_g[b])

    def wait_gathers(b):
        pltpu.make_async_copy(
            out_hbm.at[pl.ds(0, CHB)], rows_v.at[b], sem_g[b]).wait()

    def wait_out(b):
        pltpu.make_async_copy(
            rows_v.at[b], out_hbm.at[pl.ds(0, CHB)], sem_o[b]).wait()

    def scale_rows(b, r, t, jr0, lane0, nrows):
        for i in range(nrows):
            jr = jr0 + i
            s = jnp.where(t[lane0 + i] == 0,
                          jnp.float32(0.0), jnp.float32(SCALE))
            sp = jnp.full((LANES,), s, jnp.float32)
            for c in range(D // LANES):
                x = rows_v[b, r, jr, pl.ds(c * LANES, LANES)]
                rows_v[b, r, jr, pl.ds(c * LANES, LANES)] = x * sp

    def scale_chunk(b):
        for r in range(CHB):
            def grp_body(q, c2):
                t = idx_v[b, r, pl.ds(q * LANES, LANES)]
                scale_rows(b, r, t, q * LANES, 0, LANES)
                return c2

            lax.fori_loop(0, FULL_G, grp_body, 0)
            tail = S - FULL_G * LANES
            t = idx_v[b, r, pl.ds(S - LANES, LANES)]
            scale_rows(b, r, t, S - tail, LANES - tail, tail)

    def put(g, b):
        pltpu.async_copy(rows_v.at[b], out_hbm.at[pl.ds(base + g * CHB, CHB)],
                         sem_o[b])

    fire(0, 0)

    def chunk_body(g, carry):
        def stage(bb):
            @pl.when(g + 1 < N_CHUNKS)
            def _pref():
                @pl.when(g >= 1)
                def _w():
                    wait_out(1 - bb)
                fire(g + 1, 1 - bb)

            wait_gathers(bb)
            scale_chunk(bb)
            put(g, bb)

        @pl.when(lax.rem(g, 2) == 0)
        def _b0():
            stage(0)

        @pl.when(lax.rem(g, 2) == 1)
        def _b1():
            stage(1)

        return carry

    lax.fori_loop(0, N_CHUNKS, chunk_body, 0)
    wait_out((N_CHUNKS - 1) % 2)
    wait_out(N_CHUNKS % 2)


def kernel(inp_tokens, emb_table):
    mesh = plsc.VectorSubcoreMesh(core_axis_name="c", subcore_axis_name="s")
    run = pl.kernel(
        _emb_kernel,
        mesh=mesh,
        out_type=jax.ShapeDtypeStruct((BT, S, D), jnp.float32),
        scratch_types=[
            pltpu.VMEM((NBUF, CHB, S), jnp.int32),
            pltpu.VMEM((NBUF, CHB, S, D), jnp.float32),
            pltpu.SemaphoreType.DMA,
            pltpu.SemaphoreType.DMA,
            pltpu.SemaphoreType.DMA,
            pltpu.SemaphoreType.DMA,
        ],
        compiler_params=pltpu.CompilerParams(use_tc_tiling_on_sc=False),
    )
    table_pad = with_layout_constraint(
        emb_table, Layout(major_to_minor=(0, 1), tiling=((8, 128),)))
    return run(inp_tokens, table_pad)
